# baseline (speedup 1.0000x reference)
import functools

import jax
import jax.numpy as jnp
from jax.experimental import pallas as pl
from jax.experimental.pallas import tpu as pltpu
from jax.sharding import Mesh, PartitionSpec as P

N_NODES = 10000
N_EDGES = 2048
D = 128
TILE = 1000

_TN = (((0,), (0,)), ((), ()))


def _phase_a_kernel(x0_ref, B_ref, W1_ref, b1_ref,
                    Bbf_ref, h1_ref, accT_ref, degT_ref, acc_scr, deg_scr):
    t = pl.program_id(0)
    nt = pl.num_programs(0)
    Bi = B_ref[...].astype(jnp.bfloat16)
    Bbf_ref[...] = Bi
    h1 = jnp.dot(x0_ref[...].astype(jnp.bfloat16), W1_ref[...],
                 preferred_element_type=jnp.float32) + b1_ref[...]
    h1b = h1.astype(jnp.bfloat16)
    h1_ref[...] = h1b
    contribT = jax.lax.dot_general(h1b, Bi, _TN,
                                   preferred_element_type=jnp.float32)
    ones = jnp.ones((TILE, 8), jnp.bfloat16)
    dconT = jax.lax.dot_general(ones, Bi, _TN,
                                preferred_element_type=jnp.float32)

    @pl.when(t == 0)
    def _():
        acc_scr[...] = contribT
        deg_scr[...] = dconT

    @pl.when(t != 0)
    def _():
        acc_scr[...] += contribT
        deg_scr[...] += dconT

    @pl.when(t == nt - 1)
    def _():
        accT_ref[...] = acc_scr[...]
        degT_ref[...] = deg_scr[...]


def _phase_b_kernel(Bbf_ref, h1_ref, accT_ref, degT_ref, W2_ref, b2_ref,
                    h2_ref, acc2T_ref, x1b_scr, acc_scr):
    t = pl.program_id(0)
    nt = pl.num_programs(0)

    @pl.when(t == 0)
    def _():
        deg = degT_ref[0:1, :]
        x1T = accT_ref[...] / jnp.where(deg == 0.0, 1.0, deg)
        x1b_scr[...] = jnp.transpose(x1T.astype(jnp.bfloat16))

    Bi = Bbf_ref[...]
    agg = jnp.dot(Bi, x1b_scr[...], preferred_element_type=jnp.float32)
    x0l1 = jnp.maximum(h1_ref[...].astype(jnp.float32) + agg, 0.0)
    h2 = jnp.dot(x0l1.astype(jnp.bfloat16), W2_ref[...],
                 preferred_element_type=jnp.float32) + b2_ref[...]
    h2b = h2.astype(jnp.bfloat16)
    h2_ref[...] = h2b
    contribT = jax.lax.dot_general(h2b, Bi, _TN,
                                   preferred_element_type=jnp.float32)

    @pl.when(t == 0)
    def _():
        acc_scr[...] = contribT

    @pl.when(t != 0)
    def _():
        acc_scr[...] += contribT

    @pl.when(t == nt - 1)
    def _():
        acc2T_ref[...] = acc_scr[...]


def _phase_c_kernel(Bbf_ref, h2_ref, acc2T_ref, degT_ref,
                    x0_out_ref, x1_out_ref, x1b_scr):
    t = pl.program_id(0)

    @pl.when(t == 0)
    def _():
        deg = degT_ref[0:1, :]
        x1T = acc2T_ref[...] / jnp.where(deg == 0.0, 1.0, deg)
        x1b_scr[...] = jnp.transpose(x1T.astype(jnp.bfloat16))
        x1_out_ref[...] = jnp.transpose(x1T)

    Bi = Bbf_ref[...]
    agg = jnp.dot(Bi, x1b_scr[...], preferred_element_type=jnp.float32)
    x0_out_ref[...] = jnp.maximum(h2_ref[...].astype(jnp.float32) + agg, 0.0)


def _shard_fn(n_loc, x_0, incidence_1, W1b, b1, W2b, b2):
    nt = n_loc // TILE

    Bbf, h1, accT, degT = pl.pallas_call(
        _phase_a_kernel,
        grid=(nt,),
        in_specs=[
            pl.BlockSpec((TILE, D), lambda t: (t, 0)),
            pl.BlockSpec((TILE, N_EDGES), lambda t: (t, 0)),
            pl.BlockSpec((D, D), lambda t: (0, 0)),
            pl.BlockSpec((1, D), lambda t: (0, 0)),
        ],
        out_specs=[
            pl.BlockSpec((TILE, N_EDGES), lambda t: (t, 0)),
            pl.BlockSpec((TILE, D), lambda t: (t, 0)),
            pl.BlockSpec((D, N_EDGES), lambda t: (0, 0)),
            pl.BlockSpec((8, N_EDGES), lambda t: (0, 0)),
        ],
        out_shape=[
            jax.ShapeDtypeStruct((n_loc, N_EDGES), jnp.bfloat16),
            jax.ShapeDtypeStruct((n_loc, D), jnp.bfloat16),
            jax.ShapeDtypeStruct((D, N_EDGES), jnp.float32),
            jax.ShapeDtypeStruct((8, N_EDGES), jnp.float32),
        ],
        scratch_shapes=[
            pltpu.VMEM((D, N_EDGES), jnp.float32),
            pltpu.VMEM((8, N_EDGES), jnp.float32),
        ],
    )(x_0, incidence_1, W1b, b1)

    accT = jax.lax.psum(accT, "x")
    degT = jax.lax.psum(degT, "x")

    h2, acc2T = pl.pallas_call(
        _phase_b_kernel,
        grid=(nt,),
        in_specs=[
            pl.BlockSpec((TILE, N_EDGES), lambda t: (t, 0)),
            pl.BlockSpec((TILE, D), lambda t: (t, 0)),
            pl.BlockSpec((D, N_EDGES), lambda t: (0, 0)),
            pl.BlockSpec((8, N_EDGES), lambda t: (0, 0)),
            pl.BlockSpec((D, D), lambda t: (0, 0)),
            pl.BlockSpec((1, D), lambda t: (0, 0)),
        ],
        out_specs=[
            pl.BlockSpec((TILE, D), lambda t: (t, 0)),
            pl.BlockSpec((D, N_EDGES), lambda t: (0, 0)),
        ],
        out_shape=[
            jax.ShapeDtypeStruct((n_loc, D), jnp.bfloat16),
            jax.ShapeDtypeStruct((D, N_EDGES), jnp.float32),
        ],
        scratch_shapes=[
            pltpu.VMEM((N_EDGES, D), jnp.bfloat16),
            pltpu.VMEM((D, N_EDGES), jnp.float32),
        ],
    )(Bbf, h1, accT, degT, W2b, b2)

    acc2T = jax.lax.psum(acc2T, "x")

    x0_out, x1_out = pl.pallas_call(
        _phase_c_kernel,
        grid=(nt,),
        in_specs=[
            pl.BlockSpec((TILE, N_EDGES), lambda t: (t, 0)),
            pl.BlockSpec((TILE, D), lambda t: (t, 0)),
            pl.BlockSpec((D, N_EDGES), lambda t: (0, 0)),
            pl.BlockSpec((8, N_EDGES), lambda t: (0, 0)),
        ],
        out_specs=[
            pl.BlockSpec((TILE, D), lambda t: (t, 0)),
            pl.BlockSpec((N_EDGES, D), lambda t: (0, 0)),
        ],
        out_shape=[
            jax.ShapeDtypeStruct((n_loc, D), jnp.float32),
            jax.ShapeDtypeStruct((N_EDGES, D), jnp.float32),
        ],
        scratch_shapes=[
            pltpu.VMEM((N_EDGES, D), jnp.bfloat16),
        ],
    )(Bbf, h2, acc2T, degT)

    return x0_out, x1_out


def kernel(x_0, incidence_1, W1, b1, W2, b2):
    devs = jax.devices()
    n_dev = 2 if len(devs) >= 2 else 1
    n_loc = N_NODES // n_dev
    mesh = Mesh(devs[:n_dev], ("x",))
    fn = jax.shard_map(
        functools.partial(_shard_fn, n_loc),
        mesh=mesh,
        in_specs=(P("x", None), P("x", None), P(None, None), P(None, None),
                  P(None, None), P(None, None)),
        out_specs=(P("x", None), P(None, None)),
        check_vma=False,
    )
    x0_out, x1_out = fn(x_0, incidence_1,
                        W1.astype(jnp.bfloat16), b1.reshape(1, D),
                        W2.astype(jnp.bfloat16), b2.reshape(1, D))
    return (x0_out, x1_out)


# speedup vs baseline: 6.6349x; 6.6349x over previous
import jax
import jax.numpy as jnp
from jax.experimental import pallas as pl
from jax.experimental.pallas import tpu as pltpu

N_NODES = 10000
N_EDGES = 2048
D = 128
TILE = 400
T = N_NODES // TILE
TILE2 = 2000
T2 = N_NODES // TILE2

_TN = (((0,), (0,)), ((), ()))


def _uni_kernel(x0_ref, B_ref, W1_ref, b1_ref, W2_ref, b2_ref,
                x0_out_ref, x1_out_ref,
                Bbf, h1s, h2s, accT, x1b, degsT):
    s = pl.program_id(0)

    @pl.when(s < T)
    def _phase_a():
        t = s
        rows = pl.ds(t * TILE, TILE)
        Bi = B_ref[...].astype(jnp.bfloat16)
        Bbf[rows, :] = Bi
        h1 = jnp.dot(x0_ref[...].astype(jnp.bfloat16), W1_ref[...],
                     preferred_element_type=jnp.float32) + b1_ref[...]
        h1b = h1.astype(jnp.bfloat16)
        h1s[rows, :] = h1b
        contribT = jax.lax.dot_general(h1b, Bi, _TN,
                                       preferred_element_type=jnp.float32)
        ones = jnp.ones((TILE, 8), jnp.bfloat16)
        dconT = jax.lax.dot_general(ones, Bi, _TN,
                                    preferred_element_type=jnp.float32)

        @pl.when(t == 0)
        def _():
            accT[...] = contribT
            degsT[...] = dconT

        @pl.when(t != 0)
        def _():
            accT[...] += contribT
            degsT[...] += dconT

    @pl.when((s >= T) & (s < T + T2))
    def _phase_b():
        t = s - T

        @pl.when(t == 0)
        def _():
            deg = degsT[0:1, :]
            x1T = accT[...] / jnp.where(deg == 0.0, 1.0, deg)
            x1b[...] = jnp.transpose(x1T.astype(jnp.bfloat16))

        for k in range(TILE2 // TILE):
            rows = pl.ds(t * TILE2 + k * TILE, TILE)
            Bi = Bbf[rows, :]
            agg = jnp.dot(Bi, x1b[...], preferred_element_type=jnp.float32)
            x0l1 = jnp.maximum(h1s[rows, :].astype(jnp.float32) + agg, 0.0)
            h2 = jnp.dot(x0l1.astype(jnp.bfloat16), W2_ref[...],
                         preferred_element_type=jnp.float32) + b2_ref[...]
            h2b = h2.astype(jnp.bfloat16)
            h2s[rows, :] = h2b
            contribT = jax.lax.dot_general(h2b, Bi, _TN,
                                           preferred_element_type=jnp.float32)

            @pl.when((t == 0) & (k == 0))
            def _():
                accT[...] = contribT

            @pl.when(jnp.logical_not((t == 0) & (k == 0)))
            def _():
                accT[...] += contribT

    @pl.when(s >= T + T2)
    def _phase_c():
        t = s - T - T2

        @pl.when(t == 0)
        def _():
            deg = degsT[0:1, :]
            x1T = accT[...] / jnp.where(deg == 0.0, 1.0, deg)
            x1b[...] = jnp.transpose(x1T.astype(jnp.bfloat16))
            x1_out_ref[...] = jnp.transpose(x1T)

        for k in range(TILE2 // TILE):
            rows = pl.ds(t * TILE2 + k * TILE, TILE)
            orows = pl.ds(k * TILE, TILE)
            Bi = Bbf[rows, :]
            agg = jnp.dot(Bi, x1b[...], preferred_element_type=jnp.float32)
            x0_out_ref[orows, :] = jnp.maximum(
                h2s[rows, :].astype(jnp.float32) + agg, 0.0)


def _run(x_0, incidence_1, W1, b1, W2, b2, interpret=False):
    return pl.pallas_call(
        _uni_kernel,
        grid=(T + 2 * T2,),
        in_specs=[
            pl.BlockSpec((TILE, D), lambda s: (jnp.where(s < T, s, 0), 0)),
            pl.BlockSpec((TILE, N_EDGES),
                         lambda s: (jnp.where(s < T, s, 0), 0)),
            pl.BlockSpec((D, D), lambda s: (0, 0)),
            pl.BlockSpec((1, D), lambda s: (0, 0)),
            pl.BlockSpec((D, D), lambda s: (0, 0)),
            pl.BlockSpec((1, D), lambda s: (0, 0)),
        ],
        out_specs=[
            pl.BlockSpec((TILE2, D),
                         lambda s: (jnp.where(s >= T + T2, s - T - T2, 0), 0)),
            pl.BlockSpec((N_EDGES, D), lambda s: (0, 0)),
        ],
        out_shape=[
            jax.ShapeDtypeStruct((N_NODES, D), jnp.float32),
            jax.ShapeDtypeStruct((N_EDGES, D), jnp.float32),
        ],
        scratch_shapes=[
            pltpu.VMEM((N_NODES, N_EDGES), jnp.bfloat16),
            pltpu.VMEM((N_NODES, D), jnp.bfloat16),
            pltpu.VMEM((N_NODES, D), jnp.bfloat16),
            pltpu.VMEM((D, N_EDGES), jnp.float32),
            pltpu.VMEM((N_EDGES, D), jnp.bfloat16),
            pltpu.VMEM((8, N_EDGES), jnp.float32),
        ],
        interpret=interpret,
    )(x_0, incidence_1, W1.astype(jnp.bfloat16), b1.reshape(1, D),
      W2.astype(jnp.bfloat16), b2.reshape(1, D))


def kernel(x_0, incidence_1, W1, b1, W2, b2):
    x0_out, x1_out = _run(x_0, incidence_1, W1, b1, W2, b2)
    return (x0_out, x1_out)


# speedup vs baseline: 6.9581x; 1.0487x over previous
import jax
import jax.numpy as jnp
from jax.experimental import pallas as pl
from jax.experimental.pallas import tpu as pltpu

N_NODES = 10000
N_EDGES = 2048
D = 128
TILE = 400
T = N_NODES // TILE
TILE2 = 2000
T2 = N_NODES // TILE2

F8 = jnp.float8_e4m3fn
_CN = (((1,), (0,)), ((), ()))


def _hilo(x, axis):
    hi = x.astype(F8)
    lo = (x - hi.astype(jnp.float32)).astype(F8)
    return jnp.concatenate([hi, lo], axis=axis)


def _uni_kernel(x0_ref, B_ref, W1_ref, b1_ref, W2_ref, b2_ref,
                x0_out_ref, x1_out_ref,
                B8, h1s, h2s, accT, x1p, degsT):
    s = pl.program_id(0)

    @pl.when(s < T)
    def _phase_a():
        t = s
        rows = pl.ds(t * TILE, TILE)
        Bi = B_ref[...].astype(F8)
        B8[rows, :] = Bi
        h1 = jnp.dot(x0_ref[...].astype(jnp.bfloat16), W1_ref[...],
                     preferred_element_type=jnp.float32) + b1_ref[...]
        h1s[rows, :] = h1.astype(jnp.bfloat16)
        contribT = jax.lax.dot_general(_hilo(jnp.transpose(h1), 0), Bi, _CN,
                                       preferred_element_type=jnp.float32)
        ones = jnp.ones((8, TILE), F8)
        dconT = jax.lax.dot_general(ones, Bi, _CN,
                                    preferred_element_type=jnp.float32)

        @pl.when(t == 0)
        def _():
            accT[...] = contribT
            degsT[...] = dconT

        @pl.when(t != 0)
        def _():
            accT[...] += contribT
            degsT[...] += dconT

    @pl.when((s >= T) & (s < T + T2))
    def _phase_b():
        t = s - T

        @pl.when(t == 0)
        def _():
            deg = degsT[0:1, :]
            x1T = (accT[0:D, :] + accT[D:, :]) / jnp.where(deg == 0.0, 1.0, deg)
            x1p[...] = _hilo(jnp.transpose(x1T), 1)

        for k in range(TILE2 // TILE):
            rows = pl.ds(t * TILE2 + k * TILE, TILE)
            Bi = B8[rows, :]
            agg2 = jnp.dot(Bi, x1p[...], preferred_element_type=jnp.float32)
            agg = agg2[:, 0:D] + agg2[:, D:]
            x0l1 = jnp.maximum(h1s[rows, :].astype(jnp.float32) + agg, 0.0)
            h2 = jnp.dot(x0l1.astype(jnp.bfloat16), W2_ref[...],
                         preferred_element_type=jnp.float32) + b2_ref[...]
            h2s[rows, :] = h2.astype(jnp.bfloat16)
            contribT = jax.lax.dot_general(_hilo(jnp.transpose(h2), 0), Bi,
                                           _CN,
                                           preferred_element_type=jnp.float32)

            @pl.when((t == 0) & (k == 0))
            def _():
                accT[...] = contribT

            @pl.when(jnp.logical_not((t == 0) & (k == 0)))
            def _():
                accT[...] += contribT

    @pl.when(s >= T + T2)
    def _phase_c():
        t = s - T - T2

        @pl.when(t == 0)
        def _():
            deg = degsT[0:1, :]
            x1T = (accT[0:D, :] + accT[D:, :]) / jnp.where(deg == 0.0, 1.0, deg)
            x1f = jnp.transpose(x1T)
            x1p[...] = _hilo(x1f, 1)
            x1_out_ref[...] = x1f

        for k in range(TILE2 // TILE):
            rows = pl.ds(t * TILE2 + k * TILE, TILE)
            orows = pl.ds(k * TILE, TILE)
            Bi = B8[rows, :]
            agg2 = jnp.dot(Bi, x1p[...], preferred_element_type=jnp.float32)
            agg = agg2[:, 0:D] + agg2[:, D:]
            x0_out_ref[orows, :] = jnp.maximum(
                h2s[rows, :].astype(jnp.float32) + agg, 0.0)


def _run(x_0, incidence_1, W1, b1, W2, b2, interpret=False):
    return pl.pallas_call(
        _uni_kernel,
        grid=(T + 2 * T2,),
        in_specs=[
            pl.BlockSpec((TILE, D), lambda s: (jnp.where(s < T, s, 0), 0)),
            pl.BlockSpec((TILE, N_EDGES),
                         lambda s: (jnp.where(s < T, s, 0), 0)),
            pl.BlockSpec((D, D), lambda s: (0, 0)),
            pl.BlockSpec((1, D), lambda s: (0, 0)),
            pl.BlockSpec((D, D), lambda s: (0, 0)),
            pl.BlockSpec((1, D), lambda s: (0, 0)),
        ],
        out_specs=[
            pl.BlockSpec((TILE2, D),
                         lambda s: (jnp.where(s >= T + T2, s - T - T2, 0), 0)),
            pl.BlockSpec((N_EDGES, D), lambda s: (0, 0)),
        ],
        out_shape=[
            jax.ShapeDtypeStruct((N_NODES, D), jnp.float32),
            jax.ShapeDtypeStruct((N_EDGES, D), jnp.float32),
        ],
        scratch_shapes=[
            pltpu.VMEM((N_NODES, N_EDGES), F8),
            pltpu.VMEM((N_NODES, D), jnp.bfloat16),
            pltpu.VMEM((N_NODES, D), jnp.bfloat16),
            pltpu.VMEM((2 * D, N_EDGES), jnp.float32),
            pltpu.VMEM((N_EDGES, 2 * D), F8),
            pltpu.VMEM((8, N_EDGES), jnp.float32),
        ],
        interpret=interpret,
    )(x_0, incidence_1, W1.astype(jnp.bfloat16), b1.reshape(1, D),
      W2.astype(jnp.bfloat16), b2.reshape(1, D))


def kernel(x_0, incidence_1, W1, b1, W2, b2):
    x0_out, x1_out = _run(x_0, incidence_1, W1, b1, W2, b2)
    return (x0_out, x1_out)


# speedup vs baseline: 7.1598x; 1.0290x over previous
import jax
import jax.numpy as jnp
from jax.experimental import pallas as pl
from jax.experimental.pallas import tpu as pltpu

N_NODES = 10000
N_EDGES = 2048
D = 128
TILE = 400
T = N_NODES // TILE
TILE2 = 2000
T2 = N_NODES // TILE2

F8 = jnp.float8_e4m3fn
_CN = (((1,), (0,)), ((), ()))


def _hilo(x, axis):
    c = x * jnp.float32(1 << 20)
    hi = (x + c) - c
    lo = x - hi
    return jnp.concatenate([hi.astype(F8), lo.astype(F8)], axis=axis)


def _uni_kernel(x0_ref, B_ref, W1_ref, b1_ref, W2_ref, b2_ref,
                x0_out_ref, x1_out_ref,
                B8, h1s, h2s, accT, x1p, degsT):
    s = pl.program_id(0)

    @pl.when(s < T)
    def _phase_a():
        t = s
        rows = pl.ds(t * TILE, TILE)
        Bi = B_ref[...].astype(F8)
        B8[rows, :] = Bi
        h1 = jnp.dot(x0_ref[...].astype(jnp.bfloat16), W1_ref[...],
                     preferred_element_type=jnp.float32) + b1_ref[...]
        h1s[rows, :] = h1.astype(jnp.bfloat16)
        contribT = jax.lax.dot_general(_hilo(jnp.transpose(h1), 0), Bi, _CN,
                                       preferred_element_type=jnp.float32)
        ones = jnp.ones((8, TILE), F8)
        dconT = jax.lax.dot_general(ones, Bi, _CN,
                                    preferred_element_type=jnp.float32)

        @pl.when(t == 0)
        def _():
            accT[...] = jnp.zeros_like(accT)
            degsT[...] = jnp.zeros_like(degsT)

        accT[...] += contribT
        degsT[...] += dconT

    @pl.when((s >= T) & (s < T + T2))
    def _phase_b():
        t = s - T

        @pl.when(t == 0)
        def _():
            deg = degsT[0:1, :]
            x1T = (accT[0:D, :] + accT[D:, :]) / jnp.where(deg == 0.0, 1.0, deg)
            x1p[...] = _hilo(jnp.transpose(x1T), 1)
            accT[...] = jnp.zeros_like(accT)

        for k in range(TILE2 // TILE):
            rows = pl.ds(t * TILE2 + k * TILE, TILE)
            Bi = B8[rows, :]
            agg2 = jnp.dot(Bi, x1p[...], preferred_element_type=jnp.float32)
            agg = agg2[:, 0:D] + agg2[:, D:]
            x0l1 = jnp.maximum(h1s[rows, :].astype(jnp.float32) + agg, 0.0)
            h2 = jnp.dot(x0l1.astype(jnp.bfloat16), W2_ref[...],
                         preferred_element_type=jnp.float32) + b2_ref[...]
            h2s[rows, :] = h2.astype(jnp.bfloat16)
            contribT = jax.lax.dot_general(_hilo(jnp.transpose(h2), 0), Bi,
                                           _CN,
                                           preferred_element_type=jnp.float32)
            accT[...] += contribT

    @pl.when(s >= T + T2)
    def _phase_c():
        t = s - T - T2

        @pl.when(t == 0)
        def _():
            deg = degsT[0:1, :]
            x1T = (accT[0:D, :] + accT[D:, :]) / jnp.where(deg == 0.0, 1.0, deg)
            x1f = jnp.transpose(x1T)
            x1p[...] = _hilo(x1f, 1)
            x1_out_ref[...] = x1f

        for k in range(TILE2 // TILE):
            rows = pl.ds(t * TILE2 + k * TILE, TILE)
            orows = pl.ds(k * TILE, TILE)
            Bi = B8[rows, :]
            agg2 = jnp.dot(Bi, x1p[...], preferred_element_type=jnp.float32)
            agg = agg2[:, 0:D] + agg2[:, D:]
            x0_out_ref[orows, :] = jnp.maximum(
                h2s[rows, :].astype(jnp.float32) + agg, 0.0)


def _run(x_0, incidence_1, W1, b1, W2, b2, interpret=False):
    return pl.pallas_call(
        _uni_kernel,
        grid=(T + 2 * T2,),
        in_specs=[
            pl.BlockSpec((TILE, D), lambda s: (jnp.where(s < T, s, 0), 0)),
            pl.BlockSpec((TILE, N_EDGES),
                         lambda s: (jnp.where(s < T, s, 0), 0)),
            pl.BlockSpec((D, D), lambda s: (0, 0)),
            pl.BlockSpec((1, D), lambda s: (0, 0)),
            pl.BlockSpec((D, D), lambda s: (0, 0)),
            pl.BlockSpec((1, D), lambda s: (0, 0)),
        ],
        out_specs=[
            pl.BlockSpec((TILE2, D),
                         lambda s: (jnp.where(s >= T + T2, s - T - T2, 0), 0)),
            pl.BlockSpec((N_EDGES, D), lambda s: (0, 0)),
        ],
        out_shape=[
            jax.ShapeDtypeStruct((N_NODES, D), jnp.float32),
            jax.ShapeDtypeStruct((N_EDGES, D), jnp.float32),
        ],
        scratch_shapes=[
            pltpu.VMEM((N_NODES, N_EDGES), F8),
            pltpu.VMEM((N_NODES, D), jnp.bfloat16),
            pltpu.VMEM((N_NODES, D), jnp.bfloat16),
            pltpu.VMEM((2 * D, N_EDGES), jnp.float32),
            pltpu.VMEM((N_EDGES, 2 * D), F8),
            pltpu.VMEM((8, N_EDGES), jnp.float32),
        ],
        interpret=interpret,
    )(x_0, incidence_1, W1.astype(jnp.bfloat16), b1.reshape(1, D),
      W2.astype(jnp.bfloat16), b2.reshape(1, D))


def kernel(x_0, incidence_1, W1, b1, W2, b2):
    x0_out, x1_out = _run(x_0, incidence_1, W1, b1, W2, b2)
    return (x0_out, x1_out)


# speedup vs baseline: 7.3796x; 1.0307x over previous
import jax
import jax.numpy as jnp
from jax.experimental import pallas as pl
from jax.experimental.pallas import tpu as pltpu

N_NODES = 10000
N_EDGES = 2048
D = 128
TILE = 400
T = N_NODES // TILE
TILE2 = 2000
T2 = N_NODES // TILE2

F8 = jnp.float8_e4m3fn
_CN = (((1,), (0,)), ((), ()))


def _hilo(x, axis):
    c = x * jnp.float32(1 << 20)
    hi = (x + c) - c
    lo = x - hi
    return jnp.concatenate([hi.astype(F8), lo.astype(F8)], axis=axis)


def _uni_kernel(x0_ref, B_ref, W1_ref, b1_ref, W2_ref, b2_ref,
                x0_out_ref, x1_out_ref,
                B8, h1s, h2s, accT, x1p, degsT):
    s = pl.program_id(0)

    @pl.when(s < T)
    def _phase_a():
        t = s
        rows = pl.ds(t * TILE, TILE)
        Bi = B_ref[...].astype(F8)
        B8[rows, :] = Bi
        h1 = jnp.dot(x0_ref[...].astype(jnp.bfloat16), W1_ref[...],
                     preferred_element_type=jnp.float32) + b1_ref[...]
        h1s[rows, :] = h1.astype(jnp.bfloat16)
        contribT = jax.lax.dot_general(_hilo(jnp.transpose(h1), 0), Bi, _CN,
                                       preferred_element_type=jnp.float32)
        ones = jnp.ones((8, TILE), F8)
        dconT = jax.lax.dot_general(ones, Bi, _CN,
                                    preferred_element_type=jnp.float32)

        @pl.when(t == 0)
        def _():
            accT[...] = jnp.zeros_like(accT)
            degsT[...] = jnp.zeros_like(degsT)

        accT[...] += contribT[0:D, :] + contribT[D:, :]
        degsT[...] += dconT

    @pl.when((s >= T) & (s < T + T2))
    def _phase_b():
        t = s - T

        @pl.when(t == 0)
        def _():
            deg = degsT[0:1, :]
            x1T = accT[...] / jnp.where(deg == 0.0, 1.0, deg)
            x1p[...] = _hilo(jnp.transpose(x1T), 1)
            accT[...] = jnp.zeros_like(accT)

        for k in range(TILE2 // TILE):
            rows = pl.ds(t * TILE2 + k * TILE, TILE)
            Bi = B8[rows, :]
            agg2 = jnp.dot(Bi, x1p[...], preferred_element_type=jnp.float32)
            agg = agg2[:, 0:D] + agg2[:, D:]
            x0l1 = jnp.maximum(h1s[rows, :].astype(jnp.float32) + agg, 0.0)
            h2 = jnp.dot(x0l1.astype(jnp.bfloat16), W2_ref[...],
                         preferred_element_type=jnp.float32) + b2_ref[...]
            h2s[rows, :] = h2.astype(jnp.bfloat16)
            contribT = jax.lax.dot_general(_hilo(jnp.transpose(h2), 0), Bi,
                                           _CN,
                                           preferred_element_type=jnp.float32)
            accT[...] += contribT[0:D, :] + contribT[D:, :]

    @pl.when(s >= T + T2)
    def _phase_c():
        t = s - T - T2

        @pl.when(t == 0)
        def _():
            deg = degsT[0:1, :]
            x1T = accT[...] / jnp.where(deg == 0.0, 1.0, deg)
            x1f = jnp.transpose(x1T)
            x1p[...] = _hilo(x1f, 1)
            x1_out_ref[...] = x1f

        for k in range(TILE2 // TILE):
            rows = pl.ds(t * TILE2 + k * TILE, TILE)
            orows = pl.ds(k * TILE, TILE)
            Bi = B8[rows, :]
            agg2 = jnp.dot(Bi, x1p[...], preferred_element_type=jnp.float32)
            agg = agg2[:, 0:D] + agg2[:, D:]
            x0_out_ref[orows, :] = jnp.maximum(
                h2s[rows, :].astype(jnp.float32) + agg, 0.0)


def _run(x_0, incidence_1, W1, b1, W2, b2, interpret=False):
    return pl.pallas_call(
        _uni_kernel,
        grid=(T + 2 * T2,),
        in_specs=[
            pl.BlockSpec((TILE, D), lambda s: (jnp.where(s < T, s, 0), 0)),
            pl.BlockSpec((TILE, N_EDGES),
                         lambda s: (jnp.where(s < T, s, 0), 0)),
            pl.BlockSpec((D, D), lambda s: (0, 0)),
            pl.BlockSpec((1, D), lambda s: (0, 0)),
            pl.BlockSpec((D, D), lambda s: (0, 0)),
            pl.BlockSpec((1, D), lambda s: (0, 0)),
        ],
        out_specs=[
            pl.BlockSpec((TILE2, D),
                         lambda s: (jnp.where(s >= T + T2, s - T - T2, 0), 0)),
            pl.BlockSpec((N_EDGES, D), lambda s: (0, 0)),
        ],
        out_shape=[
            jax.ShapeDtypeStruct((N_NODES, D), jnp.float32),
            jax.ShapeDtypeStruct((N_EDGES, D), jnp.float32),
        ],
        scratch_shapes=[
            pltpu.VMEM((N_NODES, N_EDGES), F8),
            pltpu.VMEM((N_NODES, D), jnp.bfloat16),
            pltpu.VMEM((N_NODES, D), jnp.bfloat16),
            pltpu.VMEM((D, N_EDGES), jnp.float32),
            pltpu.VMEM((N_EDGES, 2 * D), F8),
            pltpu.VMEM((8, N_EDGES), jnp.float32),
        ],
        interpret=interpret,
    )(x_0, incidence_1, W1.astype(jnp.bfloat16), b1.reshape(1, D),
      W2.astype(jnp.bfloat16), b2.reshape(1, D))


def kernel(x_0, incidence_1, W1, b1, W2, b2):
    x0_out, x1_out = _run(x_0, incidence_1, W1, b1, W2, b2)
    return (x0_out, x1_out)
